# baseline (speedup 1.0000x reference)
import functools

import jax
import jax.numpy as jnp
from jax import lax
from jax.experimental import pallas as pl
from jax.experimental.pallas import tpu as pltpu
from jax.experimental.pallas import tpu_sc as plsc

MOM = 0.9
B = 16
K = 4096
C = 256
L = 16
NC = 2
NS = 16
NW = NC * NS
KPW = K // NW
CK = 8
NCHUNK = KPW // CK
CV = C // L


def _sc_update(proto_batch, prototypes, init_f):
    mesh = plsc.VectorSubcoreMesh(
        core_axis_name="c", subcore_axis_name="s", num_cores=NC, num_subcores=NS
    )

    @functools.partial(
        pl.kernel,
        out_type=jax.ShapeDtypeStruct((K, C), jnp.float32),
        mesh=mesh,
        compiler_params=pltpu.CompilerParams(needs_layout_passes=False),
        scratch_types=[
            pltpu.VMEM((B, CK, C), jnp.float32),
            pltpu.VMEM((CK, C), jnp.float32),
            pltpu.VMEM((CK, C), jnp.float32),
            pltpu.VMEM((KPW,), jnp.float32),
            pltpu.SemaphoreType.DMA,
        ],
    )
    def kern(pb_hbm, proto_hbm, init_hbm, out_hbm, inbuf, pbuf, obuf, ibuf, sem):
        wid = lax.axis_index("s") * NC + lax.axis_index("c")
        kbase = wid * KPW
        pltpu.sync_copy(init_hbm.at[pl.ds(kbase, KPW)], ibuf)

        @pl.loop(0, NCHUNK)
        def _chunk(ch):
            k0 = kbase + ch * CK
            descs = [
                pltpu.async_copy(pb_hbm.at[b, pl.ds(k0, CK), :], inbuf.at[b], sem)
                for b in range(B)
            ]
            descs.append(pltpu.async_copy(proto_hbm.at[pl.ds(k0, CK), :], pbuf, sem))
            for d in descs:
                d.wait()

            @pl.loop(0, CK)
            def _cls(kk):
                accs = [jnp.zeros((L,), jnp.float32) for _ in range(CV)]
                cnt = jnp.zeros((L,), jnp.float32)
                for b in range(B):
                    xs = [inbuf[b, kk, pl.ds(L * i, L)] for i in range(CV)]
                    ssp = xs[0] * xs[0]
                    for i in range(1, CV):
                        ssp = ssp + xs[i] * xs[i]
                    m = (jnp.max(ssp) > 0.0).astype(jnp.float32)
                    cnt = cnt + m
                    accs = [accs[i] + xs[i] * m for i in range(CV)]

                inv = jnp.float32(1.0) / jnp.maximum(cnt, jnp.float32(1.0))
                has_any = cnt > 0.0
                kidx = jnp.full((L,), ch * CK + kk, jnp.int32)
                a = plsc.load_gather(ibuf, [kidx]) * jnp.float32(MOM)
                for i in range(CV):
                    mean_i = accs[i] * inv
                    p_i = pbuf[kk, pl.ds(L * i, L)]
                    upd_i = mean_i + a * (p_i - mean_i)
                    obuf[kk, pl.ds(L * i, L)] = jnp.where(has_any, upd_i, p_i)

            pltpu.sync_copy(obuf, out_hbm.at[pl.ds(k0, CK), :])

    return kern(proto_batch, prototypes, init_f)


def kernel(proto_batch, prototypes, initialized):
    return _sc_update(proto_batch, prototypes, initialized.astype(jnp.float32))


# speedup vs baseline: 1.4400x; 1.4400x over previous
import functools

import jax
import jax.numpy as jnp
from jax import lax
from jax.experimental import pallas as pl
from jax.experimental.pallas import tpu as pltpu
from jax.experimental.pallas import tpu_sc as plsc

MOM = 0.9
B = 16
K = 4096
C = 256
L = 16
NC = 2
NS = 16
NW = NC * NS
KPW = K // NW
CK = 8
NCHUNK = KPW // CK
CV = C // L


def _sc_update(proto_batch, prototypes, init_f):
    mesh = plsc.VectorSubcoreMesh(
        core_axis_name="c", subcore_axis_name="s", num_cores=NC, num_subcores=NS
    )

    @functools.partial(
        pl.kernel,
        out_type=jax.ShapeDtypeStruct((K, C), jnp.float32),
        mesh=mesh,
        compiler_params=pltpu.CompilerParams(needs_layout_passes=False),
        scratch_types=[
            pltpu.VMEM((2, B, CK, C), jnp.float32),
            pltpu.VMEM((2, CK, C), jnp.float32),
            pltpu.VMEM((2, CK, C), jnp.float32),
            pltpu.VMEM((KPW,), jnp.float32),
            pltpu.SemaphoreType.DMA((2,)),
            pltpu.SemaphoreType.DMA((2,)),
        ],
    )
    def kern(pb_hbm, proto_hbm, init_hbm, out_hbm, inbuf, pbuf, obuf, ibuf,
             insem, outsem):
        wid = lax.axis_index("s") * NC + lax.axis_index("c")
        kbase = wid * KPW
        pltpu.sync_copy(init_hbm.at[pl.ds(kbase, KPW)], ibuf)

        def issue_in(slot, ch):
            k0 = kbase + ch * CK
            pltpu.async_copy(
                pb_hbm.at[:, pl.ds(k0, CK), :], inbuf.at[slot], insem.at[slot]
            )
            pltpu.async_copy(
                proto_hbm.at[pl.ds(k0, CK), :], pbuf.at[slot], insem.at[slot]
            )

        def wait_in(slot):
            pltpu.make_async_copy(
                pb_hbm.at[:, pl.ds(kbase, CK), :], inbuf.at[slot], insem.at[slot]
            ).wait()
            pltpu.make_async_copy(
                proto_hbm.at[pl.ds(kbase, CK), :], pbuf.at[slot], insem.at[slot]
            ).wait()

        def issue_out(slot, ch):
            k0 = kbase + ch * CK
            pltpu.async_copy(
                obuf.at[slot], out_hbm.at[pl.ds(k0, CK), :], outsem.at[slot]
            )

        def wait_out(slot):
            pltpu.make_async_copy(
                obuf.at[slot], out_hbm.at[pl.ds(kbase, CK), :], outsem.at[slot]
            ).wait()

        def compute(slot, ch):
            @pl.loop(0, CK)
            def _cls(kk):
                accs = [jnp.zeros((L,), jnp.float32) for _ in range(CV)]
                cnt = jnp.zeros((L,), jnp.float32)
                for b in range(B):
                    xs = [inbuf[slot, b, kk, pl.ds(L * i, L)] for i in range(CV)]
                    sq = [xs[i] * xs[i] for i in range(CV)]
                    while len(sq) > 1:
                        sq = [sq[j] + sq[j + 1] for j in range(0, len(sq) - 1, 2)] + (
                            [sq[-1]] if len(sq) % 2 else []
                        )
                    m = (jnp.max(sq[0]) > 0.0).astype(jnp.float32)
                    cnt = cnt + m
                    accs = [accs[i] + xs[i] for i in range(CV)]

                inv = jnp.float32(1.0) / jnp.maximum(cnt, jnp.float32(1.0))
                has_any = cnt > 0.0
                kidx = jnp.full((L,), ch * CK + kk, jnp.int32)
                a = plsc.load_gather(ibuf, [kidx]) * jnp.float32(MOM)
                for i in range(CV):
                    mean_i = accs[i] * inv
                    p_i = pbuf[slot, kk, pl.ds(L * i, L)]
                    upd_i = mean_i + a * (p_i - mean_i)
                    obuf[slot, kk, pl.ds(L * i, L)] = jnp.where(has_any, upd_i, p_i)

        issue_in(0, 0)

        @pl.loop(0, NCHUNK, step=2)
        def _chunk(ch):
            issue_in(1, ch + 1)
            wait_in(0)

            @pl.when(ch >= 2)
            def _():
                wait_out(0)

            compute(0, ch)
            issue_out(0, ch)

            @pl.when(ch + 2 < NCHUNK)
            def _():
                issue_in(0, ch + 2)

            wait_in(1)

            @pl.when(ch >= 2)
            def _():
                wait_out(1)

            compute(1, ch + 1)
            issue_out(1, ch + 1)

        wait_out(0)
        wait_out(1)

    return kern(proto_batch, prototypes, init_f)


def kernel(proto_batch, prototypes, initialized):
    return _sc_update(proto_batch, prototypes, initialized.astype(jnp.float32))
